# baseline (speedup 1.0000x reference)
import functools

import jax
import jax.numpy as jnp
from jax import lax
from jax.experimental import pallas as pl
from jax.experimental.pallas import tpu as pltpu
from jax.experimental.pallas import tpu_sc as plsc

B = 64
N2 = 361
RPAD = 368
ZPAD = 384
MPAD = 3712
NG = 91
FLATB = 131072
FLATW = FLATB // 4

_NW = 32


def _sc_body(cap_hbm, zt_hbm, scal_hbm, nh_hbm, mvm, ztv, dpl, dop, tbl,
             scv, nhv):
    wid = lax.axis_index("s") * 2 + lax.axis_index("c")
    pltpu.sync_copy(zt_hbm, ztv)
    lanes = lax.iota(jnp.int32, 16)
    capw = cap_hbm.bitcast(jnp.int32)

    def one_batch(g, _):
        b = wid * 2 + g
        pltpu.sync_copy(capw.at[pl.ds(b * 256, 256)], mvm)
        pltpu.sync_copy(scal_hbm.at[b], scv)
        pv = scv[pl.ds(0, 16)]
        hashv = scv[pl.ds(16, 16)]
        is_p0 = pv == 0

        def build_d(i, _):
            z0 = ztv[pl.ds(i * 16, 16)]
            z1 = ztv[pl.ds(ZPAD + i * 16, 16)]
            z2 = ztv[pl.ds(2 * ZPAD + i * 16, 16)]
            d1 = z0 ^ z1
            d2 = z0 ^ z2
            dpl[pl.ds(i * 16, 16)] = jnp.where(is_p0, d1, d2)
            dop[pl.ds(i * 16, 16)] = jnp.where(is_p0, d2, d1)
            return 0

        lax.fori_loop(0, ZPAD // 16, build_d, 0)

        def build_t(G, _):
            gidx = G * 16 + lanes
            base = gidx * 16
            d = [plsc.load_gather(dop, [gidx * 4 + k]) for k in range(4)]
            vals = [jnp.zeros((16,), jnp.int32)] * 16
            for s in range(1, 16):
                lowk = (s & -s).bit_length() - 1
                vals[s] = vals[s & (s - 1)] ^ d[lowk]
            for s in range(16):
                plsc.store_scatter(tbl, [base + s], vals[s])
            return 0

        lax.fori_loop(0, 6, build_t, 0)

        def row_chunk(r, _):
            rows = jnp.minimum(r * 16 + lanes, N2 - 1)
            pbase = rows * N2
            q = pbase >> 2
            sh = (pbase & 3) * 8
            shc = 31 - sh

            def t_step(t, carry):
                acc, g_cur = carry
                wv = q + t + 1
                g_next = plsc.load_gather(mvm, [wv >> 7, wv & 127])
                al = lax.shift_right_logical(g_cur, sh) | (
                    (g_next << shc) << 1)
                idx = lax.shift_right_logical(al * 0x00204081, 21) & 15
                acc = acc ^ plsc.load_gather(tbl, [idx + t * 16])
                return acc, g_next

            cap, _ = lax.fori_loop(
                0, NG, t_step,
                (jnp.zeros((16,), jnp.int32),
                 plsc.load_gather(mvm, [q >> 7, q & 127])),
                unroll=7)
            pd = dpl[pl.ds(r * 16, 16)]
            nhv[pl.ds(r * 16, 16)] = hashv ^ pd ^ cap
            return 0

        lax.fori_loop(0, RPAD // 16, row_chunk, 0)
        pltpu.sync_copy(nhv, nh_hbm.at[b])
        return 0

    lax.fori_loop(0, 2, one_batch, 0)


_sc_hash = functools.partial(
    pl.kernel,
    out_type=jax.ShapeDtypeStruct((B, RPAD), jnp.int32),
    mesh=plsc.VectorSubcoreMesh(core_axis_name="c", subcore_axis_name="s"),
    scratch_types=[
        pltpu.VMEM((256, 128), jnp.int32),
        pltpu.VMEM((3 * ZPAD,), jnp.int32),
        pltpu.VMEM((ZPAD,), jnp.int32),
        pltpu.VMEM((ZPAD + 16,), jnp.int32),
        pltpu.VMEM((96 * 16,), jnp.int32),
        pltpu.VMEM((32,), jnp.int32),
        pltpu.VMEM((RPAD,), jnp.int32),
    ],
    compiler_params=pltpu.CompilerParams(needs_layout_passes=False),
)(_sc_body)


CH = 128


def _tc_body(nh_ref, hist_ref, mc_ref, legal_ref, out_ref):
    mc = mc_ref[pl.program_id(0), 0]
    nh = nh_ref[0]

    def chunk(c, rep):
        h = hist_ref[0, :, pl.ds(c * CH, CH)]
        vi = lax.broadcasted_iota(jnp.int32, (1, CH), 1) + c * CH
        hm = jnp.where(vi < mc, h, -1)
        return rep | (nh == hm).astype(jnp.int32)

    nchunks = (mc + (CH - 1)) // CH
    rep = lax.fori_loop(0, nchunks, chunk, jnp.zeros((RPAD, CH), jnp.int32))
    hit = jnp.any(rep != 0, axis=1, keepdims=True)
    out_ref[0] = legal_ref[0] & jnp.where(hit, 0, 1)


def _tc_member(nh_col, hist_row, mc, legal_col):
    return pl.pallas_call(
        _tc_body,
        grid=(B,),
        in_specs=[
            pl.BlockSpec((1, RPAD, 1), lambda b: (b, 0, 0)),
            pl.BlockSpec((1, 1, MPAD), lambda b: (b, 0, 0)),
            pl.BlockSpec((B, 1), lambda b: (0, 0), memory_space=pltpu.SMEM),
            pl.BlockSpec((1, RPAD, 1), lambda b: (b, 0, 0)),
        ],
        out_specs=pl.BlockSpec((1, RPAD, 1), lambda b: (b, 0, 0)),
        out_shape=jax.ShapeDtypeStruct((B, RPAD, 1), jnp.int32),
    )(nh_col, hist_row, mc, legal_col)


def kernel(legal_mask, capture_stone_mask, current_player, current_hash,
           hash_history, move_count, Zpos):
    Bq, H, W = legal_mask.shape

    cap8 = capture_stone_mask.astype(jnp.uint8).reshape(B, N2 * N2)
    cap8 = jnp.pad(cap8, ((0, 0), (0, FLATB - N2 * N2)))
    cap8 = cap8.reshape(B, 256, 128, 4).transpose(0, 1, 3, 2).reshape(
        B * 1024, 128)

    zt = jnp.pad(Zpos.T, ((0, 0), (0, ZPAD - N2))).reshape(3 * ZPAD)
    scal = jnp.stack(
        [current_player.astype(jnp.int32), current_hash], axis=1)
    scal16 = jnp.broadcast_to(scal[:, :, None], (B, 2, 16)).reshape(B, 32)

    nh = _sc_hash(cap8, zt, scal16)

    hist_row = jnp.pad(
        hash_history, ((0, 0), (0, MPAD - hash_history.shape[1]))
    ).reshape(B, 1, MPAD)
    legal_col = jnp.pad(
        legal_mask.reshape(B, N2).astype(jnp.int32),
        ((0, 0), (0, RPAD - N2))).reshape(B, RPAD, 1)
    mc = move_count.reshape(B, 1)

    out = _tc_member(nh.reshape(B, RPAD, 1), hist_row, mc, legal_col)
    return out.reshape(B, RPAD)[:, :N2].astype(bool).reshape(B, H, W)


# speedup vs baseline: 1.0893x; 1.0893x over previous
import functools

import jax
import jax.numpy as jnp
from jax import lax
from jax.experimental import pallas as pl
from jax.experimental.pallas import tpu as pltpu
from jax.experimental.pallas import tpu_sc as plsc

B = 64
N2 = 361
RPAD = 368
ZPAD = 384
MPAD = 3712
NG = 91
FLATB = 131072
FLATW = FLATB // 4

_NW = 32


def _sc_body(cap_hbm, zt_hbm, scal_hbm, nh_hbm, mvm, ztv, dpl, dop, tbl,
             scv, nhv):
    wid = lax.axis_index("s") * 2 + lax.axis_index("c")
    pltpu.sync_copy(zt_hbm, ztv)
    lanes = lax.iota(jnp.int32, 16)
    capw = cap_hbm.bitcast(jnp.int32)

    def one_batch(g, _):
        b = wid * 2 + g
        pltpu.sync_copy(capw.at[pl.ds(b * 256, 256)], mvm)
        pltpu.sync_copy(scal_hbm.at[b], scv)
        pv = scv[pl.ds(0, 16)]
        hashv = scv[pl.ds(16, 16)]
        is_p0 = pv == 0

        def build_d(i, _):
            z0 = ztv[pl.ds(i * 16, 16)]
            z1 = ztv[pl.ds(ZPAD + i * 16, 16)]
            z2 = ztv[pl.ds(2 * ZPAD + i * 16, 16)]
            d1 = z0 ^ z1
            d2 = z0 ^ z2
            dpl[pl.ds(i * 16, 16)] = jnp.where(is_p0, d1, d2)
            dop[pl.ds(i * 16, 16)] = jnp.where(is_p0, d2, d1)
            return 0

        lax.fori_loop(0, ZPAD // 16, build_d, 0)

        def build_t(G, _):
            gidx = G * 16 + lanes
            base = gidx * 16
            d = [plsc.load_gather(dop, [gidx * 4 + k]) for k in range(4)]
            vals = [jnp.zeros((16,), jnp.int32)] * 16
            for s in range(1, 16):
                lowk = (s & -s).bit_length() - 1
                vals[s] = vals[s & (s - 1)] ^ d[lowk]
            for s in range(16):
                plsc.store_scatter(tbl, [base + s], vals[s])
            return 0

        lax.fori_loop(0, 6, build_t, 0)

        def row_chunk(r, _):
            rows = jnp.minimum(r * 16 + lanes, N2 - 1)
            pbase = rows * N2
            q = pbase >> 2
            sh = (pbase & 3) * 8
            shc = 31 - sh

            def t_step(t, carry):
                acc, g_cur = carry
                wv = q + t + 1
                g_next = plsc.load_gather(mvm, [wv >> 7, wv & 127])
                al = lax.shift_right_logical(g_cur, sh) | (
                    (g_next << shc) << 1)
                idx = lax.shift_right_logical(al * 0x00204081, 21) & 15
                acc = acc ^ plsc.load_gather(tbl, [idx + t * 16])
                return acc, g_next

            cap, _ = lax.fori_loop(
                0, NG, t_step,
                (jnp.zeros((16,), jnp.int32),
                 plsc.load_gather(mvm, [q >> 7, q & 127])),
                unroll=7)
            pd = dpl[pl.ds(r * 16, 16)]
            nhv[pl.ds(r * 16, 16)] = hashv ^ pd ^ cap
            return 0

        lax.fori_loop(0, RPAD // 16, row_chunk, 0)
        pltpu.sync_copy(nhv, nh_hbm.at[b])
        return 0

    lax.fori_loop(0, 2, one_batch, 0)


_sc_hash = functools.partial(
    pl.kernel,
    out_type=jax.ShapeDtypeStruct((B, RPAD), jnp.int32),
    mesh=plsc.VectorSubcoreMesh(core_axis_name="c", subcore_axis_name="s"),
    scratch_types=[
        pltpu.VMEM((256, 128), jnp.int32),
        pltpu.VMEM((3 * ZPAD,), jnp.int32),
        pltpu.VMEM((ZPAD,), jnp.int32),
        pltpu.VMEM((ZPAD + 16,), jnp.int32),
        pltpu.VMEM((96 * 16,), jnp.int32),
        pltpu.VMEM((32,), jnp.int32),
        pltpu.VMEM((RPAD,), jnp.int32),
    ],
    compiler_params=pltpu.CompilerParams(needs_layout_passes=False),
)(_sc_body)


CH = 232


def _tc_body(nh_ref, hist_ref, mc_ref, legal_ref, out_ref):
    mc = mc_ref[pl.program_id(0), 0]
    nh = nh_ref[0]

    def chunk(c, rep):
        h = hist_ref[0, pl.ds(c * CH, CH)]
        vi = lax.broadcasted_iota(jnp.int32, (CH, 1), 0) + c * CH
        hm = jnp.where(vi < mc, h, -1)
        eq = hm == nh
        return rep | jnp.any(eq, axis=0, keepdims=True).astype(jnp.int32)

    nchunks = (mc + (CH - 1)) // CH
    rep = lax.fori_loop(0, nchunks, chunk, jnp.zeros((1, RPAD), jnp.int32))
    out_ref[0] = legal_ref[0] & (1 - rep)


def _tc_member(nh, hist_col, mc, legal):
    return pl.pallas_call(
        _tc_body,
        grid=(B,),
        in_specs=[
            pl.BlockSpec((1, 1, RPAD), lambda b: (b, 0, 0)),
            pl.BlockSpec((1, MPAD, 1), lambda b: (b, 0, 0)),
            pl.BlockSpec((B, 1), lambda b: (0, 0), memory_space=pltpu.SMEM),
            pl.BlockSpec((1, 1, RPAD), lambda b: (b, 0, 0)),
        ],
        out_specs=pl.BlockSpec((1, 1, RPAD), lambda b: (b, 0, 0)),
        out_shape=jax.ShapeDtypeStruct((B, 1, RPAD), jnp.int32),
    )(nh, hist_col, mc, legal)


def kernel(legal_mask, capture_stone_mask, current_player, current_hash,
           hash_history, move_count, Zpos):
    Bq, H, W = legal_mask.shape

    cap8 = capture_stone_mask.astype(jnp.uint8).reshape(B, N2 * N2)
    cap8 = jnp.pad(cap8, ((0, 0), (0, FLATB - N2 * N2)))
    cap8 = cap8.reshape(B, 256, 128, 4).transpose(0, 1, 3, 2).reshape(
        B * 1024, 128)

    zt = jnp.pad(Zpos.T, ((0, 0), (0, ZPAD - N2))).reshape(3 * ZPAD)
    scal = jnp.stack(
        [current_player.astype(jnp.int32), current_hash], axis=1)
    scal16 = jnp.broadcast_to(scal[:, :, None], (B, 2, 16)).reshape(B, 32)

    nh = _sc_hash(cap8, zt, scal16)

    hist_col = jnp.pad(
        hash_history, ((0, 0), (0, MPAD - hash_history.shape[1]))
    ).reshape(B, MPAD, 1)
    legal_i = jnp.pad(
        legal_mask.reshape(B, N2).astype(jnp.int32),
        ((0, 0), (0, RPAD - N2))).reshape(B, 1, RPAD)
    mc = move_count.reshape(B, 1)

    out = _tc_member(nh.reshape(B, 1, RPAD), hist_col, mc, legal_i)
    return out.reshape(B, RPAD)[:, :N2].astype(bool).reshape(B, H, W)


# speedup vs baseline: 1.0900x; 1.0006x over previous
import functools

import jax
import jax.numpy as jnp
from jax import lax
from jax.experimental import pallas as pl
from jax.experimental.pallas import tpu as pltpu
from jax.experimental.pallas import tpu_sc as plsc

B = 64
N2 = 361
RPAD = 368
ZPAD = 384
MPAD = 3712
NG = 91
FLATB = 131072
FLATW = FLATB // 4


def _sc_body(cap_hbm, zt_hbm, scal_hbm, nh_hbm, mvm, ztv, dpl, dop, tbl,
             scv, nhv):
    wid = lax.axis_index("s") * 2 + lax.axis_index("c")
    pltpu.sync_copy(zt_hbm, ztv)
    lanes = lax.iota(jnp.int32, 16)
    capw = cap_hbm.bitcast(jnp.int32)

    def one_batch(g, _):
        b = wid * 2 + g
        pltpu.sync_copy(capw.at[pl.ds(b * 256, 256)], mvm)
        pltpu.sync_copy(scal_hbm.at[b], scv)
        pv = scv[pl.ds(0, 16)]
        hashv = scv[pl.ds(16, 16)]
        is_p0 = pv == 0

        def build_d(i, _):
            z0 = ztv[pl.ds(i * 16, 16)]
            z1 = ztv[pl.ds(ZPAD + i * 16, 16)]
            z2 = ztv[pl.ds(2 * ZPAD + i * 16, 16)]
            d1 = z0 ^ z1
            d2 = z0 ^ z2
            dpl[pl.ds(i * 16, 16)] = jnp.where(is_p0, d1, d2)
            dop[pl.ds(i * 16, 16)] = jnp.where(is_p0, d2, d1)
            return 0

        lax.fori_loop(0, ZPAD // 16, build_d, 0)

        def build_t(G, _):
            gidx = G * 16 + lanes
            base = gidx * 16
            d = [plsc.load_gather(dop, [gidx * 4 + k]) for k in range(4)]
            vals = [jnp.zeros((16,), jnp.int32)] * 16
            for s in range(1, 16):
                lowk = (s & -s).bit_length() - 1
                vals[s] = vals[s & (s - 1)] ^ d[lowk]
            for s in range(16):
                plsc.store_scatter(tbl, [base + s], vals[s])
            return 0

        lax.fori_loop(0, 6, build_t, 0)

        def row_chunk(r, _):
            rows = jnp.minimum(r * 16 + lanes, N2 - 1)
            pbase = rows * N2
            q = pbase >> 2
            sh = (pbase & 3) * 8
            shc = 31 - sh

            def t_step(t, carry):
                acc, g_cur = carry
                wv = q + t + 1
                g_next = plsc.load_gather(mvm, [wv >> 7, wv & 127])
                al = lax.shift_right_logical(g_cur, sh) | (
                    (g_next << shc) << 1)
                idx = lax.shift_right_logical(al * 0x00204081, 21) & 15
                acc = acc ^ plsc.load_gather(tbl, [idx + t * 16])
                return acc, g_next

            cap, _ = lax.fori_loop(
                0, NG, t_step,
                (jnp.zeros((16,), jnp.int32),
                 plsc.load_gather(mvm, [q >> 7, q & 127])),
                unroll=7)
            pd = dpl[pl.ds(r * 16, 16)]
            nhv[pl.ds(r * 16, 16)] = hashv ^ pd ^ cap
            return 0

        lax.fori_loop(0, RPAD // 16, row_chunk, 0)
        pltpu.sync_copy(nhv, nh_hbm.at[b])
        return 0

    lax.fori_loop(0, 2, one_batch, 0)


_sc_hash = functools.partial(
    pl.kernel,
    out_type=jax.ShapeDtypeStruct((B, RPAD), jnp.int32),
    mesh=plsc.VectorSubcoreMesh(core_axis_name="c", subcore_axis_name="s"),
    scratch_types=[
        pltpu.VMEM((256, 128), jnp.int32),
        pltpu.VMEM((3 * ZPAD,), jnp.int32),
        pltpu.VMEM((ZPAD,), jnp.int32),
        pltpu.VMEM((ZPAD + 16,), jnp.int32),
        pltpu.VMEM((96 * 16,), jnp.int32),
        pltpu.VMEM((32,), jnp.int32),
        pltpu.VMEM((RPAD,), jnp.int32),
    ],
    compiler_params=pltpu.CompilerParams(needs_layout_passes=False),
)(_sc_body)


CH = 232


def _tc_body(nh_ref, hist_ref, mc_ref, legal_ref, out_ref):
    mc = mc_ref[pl.program_id(0), 0]
    nh = nh_ref[0]

    def chunk(c, rep):
        h = hist_ref[0, pl.ds(c * CH, CH)]
        vi = lax.broadcasted_iota(jnp.int32, (CH, 1), 0) + c * CH
        hm = jnp.where(vi < mc, h, -1)
        eq = hm == nh
        return rep | jnp.any(eq, axis=0, keepdims=True).astype(jnp.int32)

    nchunks = (mc + (CH - 1)) // CH
    rep = lax.fori_loop(0, nchunks, chunk, jnp.zeros((1, RPAD), jnp.int32))
    out_ref[0] = legal_ref[0] & (1 - rep)


def _tc_member(nh, hist_col, mc, legal):
    return pl.pallas_call(
        _tc_body,
        grid=(B,),
        in_specs=[
            pl.BlockSpec((1, 1, RPAD), lambda b: (b, 0, 0)),
            pl.BlockSpec((1, MPAD, 1), lambda b: (b, 0, 0)),
            pl.BlockSpec((B, 1), lambda b: (0, 0), memory_space=pltpu.SMEM),
            pl.BlockSpec((1, 1, RPAD), lambda b: (b, 0, 0)),
        ],
        out_specs=pl.BlockSpec((1, 1, RPAD), lambda b: (b, 0, 0)),
        out_shape=jax.ShapeDtypeStruct((B, 1, RPAD), jnp.int32),
    )(nh, hist_col, mc, legal)


def kernel(legal_mask, capture_stone_mask, current_player, current_hash,
           hash_history, move_count, Zpos):
    Bq, H, W = legal_mask.shape

    cap8 = capture_stone_mask.astype(jnp.uint8).reshape(B, N2 * N2)
    cap8 = jnp.pad(cap8, ((0, 0), (0, FLATB - N2 * N2)))
    cap8 = cap8.reshape(B, 256, 128, 4).transpose(0, 1, 3, 2).reshape(
        B * 1024, 128)

    zt = jnp.pad(Zpos.T, ((0, 0), (0, ZPAD - N2))).reshape(3 * ZPAD)
    scal = jnp.stack(
        [current_player.astype(jnp.int32), current_hash], axis=1)
    scal16 = jnp.broadcast_to(scal[:, :, None], (B, 2, 16)).reshape(B, 32)

    nh = _sc_hash(cap8, zt, scal16)

    hist_col = jnp.pad(
        hash_history, ((0, 0), (0, MPAD - hash_history.shape[1]))
    ).reshape(B, MPAD, 1)
    legal_i = jnp.pad(
        legal_mask.reshape(B, N2).astype(jnp.int32),
        ((0, 0), (0, RPAD - N2))).reshape(B, 1, RPAD)
    mc = move_count.reshape(B, 1)

    out = _tc_member(nh.reshape(B, 1, RPAD), hist_col, mc, legal_i)
    return out.reshape(B, RPAD)[:, :N2].astype(bool).reshape(B, H, W)
